# baseline (speedup 1.0000x reference)
import functools

import jax
import jax.numpy as jnp
from jax import lax
from jax.experimental import pallas as pl
from jax.experimental.pallas import tpu as pltpu
from jax.experimental.pallas import tpu_sc as plsc

GRAPH_BIAS_SCALE = 0.8
CONTEXT_GATE_STRENGTH = 2.0
COMMITMENT_COST = 0.25

N = 8192
D = 256
BN = 1024


def _body(z_ref, cb_ref, w_ref, b_ref, idx_ref, loss_ref, cb2_ref, cn_ref):
    step = pl.program_id(0)

    @pl.when(step == 0)
    def _():
        loss_ref[0, 0] = 0.0
        cb = cb_ref[...]
        cb2_ref[...] = cb + cb
        cn_ref[...] = jnp.sum(cb * cb, axis=1)[None, :]

    z = z_ref[...]
    zn = jnp.sum(z * z, axis=1, keepdims=True)
    zc2 = jax.lax.dot_general(z, cb2_ref[...], (((1,), (1,)), ((), ())),
                              preferred_element_type=jnp.float32)
    logits = jnp.dot(z, w_ref[...], preferred_element_type=jnp.float32) + b_ref[...]
    e = jnp.exp(logits)
    g = (CONTEXT_GATE_STRENGTH / jnp.sum(e, axis=1, keepdims=True)) * e
    d = (zn + cn_ref[...] - zc2) - g
    k = d.shape[1]
    d_min = jnp.min(d, axis=1)
    eq = d == d_min[:, None]
    code = (jax.lax.broadcasted_iota(jnp.int32, d.shape, 1) * 4
            ).astype(jnp.float32) - g
    v = jnp.min(jnp.where(eq, code, 4.0 * k), axis=1)
    idx_f = jnp.floor(v * 0.25 + 0.5)
    idx_ref[...] = idx_f.astype(jnp.int32)
    g_idx = idx_f * 4.0 - v
    loss_ref[0, 0] += jnp.sum(d_min + g_idx)


def _tc_branch(z, cb, w, b):
    k = cb.shape[0]
    grid = (N // BN,)
    full = lambda shape: pl.BlockSpec(shape, lambda i: (0,) * len(shape))

    idx, losssum = pl.pallas_call(
        _body,
        grid=grid,
        in_specs=[
            pl.BlockSpec((BN, D), lambda i: (i, 0)),
            full((k, D)),
            full((D, k)),
            full((1, k)),
        ],
        out_specs=(
            pl.BlockSpec((BN,), lambda i: (i,)),
            pl.BlockSpec((1, 1), lambda i: (0, 0), memory_space=pltpu.SMEM),
        ),
        out_shape=(
            jax.ShapeDtypeStruct((N,), jnp.int32),
            jax.ShapeDtypeStruct((1, 1), jnp.float32),
        ),
        scratch_shapes=[
            pltpu.VMEM((k, D), jnp.float32),
            pltpu.VMEM((1, k), jnp.float32),
        ],
    )(z, cb, w, b.reshape(1, k))
    return idx, losssum


def _sc_gather_planes(cb_re, cb_im, idx):
    info = plsc.get_sparse_core_info()
    nw = info.num_cores * info.num_subcores
    b_per_w = N // nw
    half = D // 2
    mesh = plsc.VectorSubcoreMesh(core_axis_name="c", subcore_axis_name="s")

    @functools.partial(
        pl.kernel,
        mesh=mesh,
        out_type=(jax.ShapeDtypeStruct((N, half), jnp.float32),
                  jax.ShapeDtypeStruct((N, half), jnp.float32)),
        scratch_types=[
            pltpu.VMEM((b_per_w,), jnp.int32),
            pltpu.VMEM((b_per_w, half), jnp.float32),
            pltpu.VMEM((b_per_w, half), jnp.float32),
            pltpu.SemaphoreType.DMA,
            pltpu.SemaphoreType.DMA,
        ],
    )
    def _k(cbre_hbm, cbim_hbm, idx_hbm, outre_hbm, outim_hbm,
           idx_v, re_v, im_v, sem_re, sem_im):
        wid = lax.axis_index("s") * info.num_cores + lax.axis_index("c")
        base = wid * b_per_w
        pltpu.sync_copy(idx_hbm.at[pl.ds(base, b_per_w)], idx_v)
        cp_re = pltpu.async_copy(cbre_hbm.at[idx_v], re_v, sem_re)
        cp_im = pltpu.async_copy(cbim_hbm.at[idx_v], im_v, sem_im)
        cp_re.wait()
        pltpu.sync_copy(re_v, outre_hbm.at[pl.ds(base, b_per_w)])
        cp_im.wait()
        pltpu.sync_copy(im_v, outim_hbm.at[pl.ds(base, b_per_w)])

    return _k(cb_re, cb_im, idx)


def kernel(z_fast, z_slow, cb_syn, cb_sem, Wg_syn, bg_syn, Wg_sem, bg_sem):
    half = D // 2
    idx_syn, ls = _tc_branch(z_fast, cb_syn, Wg_syn, bg_syn)
    idx_sem, lm = _tc_branch(z_slow, cb_sem, Wg_sem, bg_sem)
    re_syn, im_syn = _sc_gather_planes(cb_syn[:, :half], cb_syn[:, half:],
                                       idx_syn)
    re_sem, im_sem = _sc_gather_planes(cb_sem[:, :half], cb_sem[:, half:],
                                       idx_sem)
    loss = (ls[0, 0] + lm[0, 0]) * ((1.0 + COMMITMENT_COST) / (N * D))
    zqc_syn = jax.lax.complex(re_syn, im_syn)
    zqc_sem = jax.lax.complex(re_sem, im_sem)
    return (zqc_syn, zqc_sem, loss, (idx_syn, idx_sem))


# speedup vs baseline: 1.2642x; 1.2642x over previous
import functools

import jax
import jax.numpy as jnp
from jax import lax
from jax.experimental import pallas as pl
from jax.experimental.pallas import tpu as pltpu
from jax.experimental.pallas import tpu_sc as plsc

GRAPH_BIAS_SCALE = 0.8
CONTEXT_GATE_STRENGTH = 2.0
COMMITMENT_COST = 0.25

N = 8192
D = 256
BN = 1024


def _body(z_ref, cb_ref, w_ref, b_ref, idx_ref, loss_ref, cb2_ref, cn_ref):
    step = pl.program_id(0)

    @pl.when(step == 0)
    def _():
        loss_ref[0, 0] = 0.0
        cb = cb_ref[...]
        cb2_ref[...] = cb + cb
        cn_ref[...] = jnp.sum(cb * cb, axis=1)[None, :]

    z = z_ref[...]
    zn = jnp.sum(z * z, axis=1, keepdims=True)
    zc2 = jax.lax.dot_general(z, cb2_ref[...], (((1,), (1,)), ((), ())),
                              preferred_element_type=jnp.float32)
    logits = jnp.dot(z, w_ref[...], preferred_element_type=jnp.float32) + b_ref[...]
    e = jnp.exp(logits)
    g = (CONTEXT_GATE_STRENGTH / jnp.sum(e, axis=1, keepdims=True)) * e
    d = (zn + cn_ref[...] - zc2) - g
    k = d.shape[1]
    d_min = jnp.min(d, axis=1)
    eq = d == d_min[:, None]
    code = (jax.lax.broadcasted_iota(jnp.int32, d.shape, 1) * 4
            ).astype(jnp.float32) - g
    v = jnp.min(jnp.where(eq, code, 4.0 * k), axis=1)
    idx_f = jnp.floor(v * 0.25 + 0.5)
    idx_ref[...] = idx_f.astype(jnp.int32)
    g_idx = idx_f * 4.0 - v
    loss_ref[0, 0] += jnp.sum(d_min + g_idx)


def _tc_branch(z, cb, w, b):
    k = cb.shape[0]
    grid = (N // BN,)
    full = lambda shape: pl.BlockSpec(shape, lambda i: (0,) * len(shape))

    idx, losssum = pl.pallas_call(
        _body,
        grid=grid,
        in_specs=[
            pl.BlockSpec((BN, D), lambda i: (i, 0)),
            full((k, D)),
            full((D, k)),
            full((1, k)),
        ],
        out_specs=(
            pl.BlockSpec((BN,), lambda i: (i,)),
            pl.BlockSpec((1, 1), lambda i: (0, 0), memory_space=pltpu.SMEM),
        ),
        out_shape=(
            jax.ShapeDtypeStruct((N,), jnp.int32),
            jax.ShapeDtypeStruct((1, 1), jnp.float32),
        ),
        scratch_shapes=[
            pltpu.VMEM((k, D), jnp.float32),
            pltpu.VMEM((1, k), jnp.float32),
        ],
    )(z, cb, w, b.reshape(1, k))
    return idx, losssum


def _sc_gather(cb, idx):
    info = plsc.get_sparse_core_info()
    nw = info.num_cores * info.num_subcores
    b_per_w = N // nw
    mesh = plsc.VectorSubcoreMesh(core_axis_name="c", subcore_axis_name="s")

    @functools.partial(
        pl.kernel,
        mesh=mesh,
        out_type=jax.ShapeDtypeStruct((N, D), jnp.float32),
        scratch_types=[
            pltpu.VMEM((b_per_w,), jnp.int32),
            pltpu.VMEM((b_per_w, D), jnp.float32),
            pltpu.SemaphoreType.DMA,
        ],
    )
    def _k(cb_hbm, idx_hbm, out_hbm, idx_v, rows_v, sem):
        wid = lax.axis_index("s") * info.num_cores + lax.axis_index("c")
        base = wid * b_per_w
        pltpu.sync_copy(idx_hbm.at[pl.ds(base, b_per_w)], idx_v)
        pltpu.async_copy(cb_hbm.at[idx_v], rows_v, sem).wait()
        pltpu.sync_copy(rows_v, out_hbm.at[pl.ds(base, b_per_w)])

    return _k(cb, idx)


def kernel(z_fast, z_slow, cb_syn, cb_sem, Wg_syn, bg_syn, Wg_sem, bg_sem):
    half = D // 2
    idx_syn, ls = _tc_branch(z_fast, cb_syn, Wg_syn, bg_syn)
    idx_sem, lm = _tc_branch(z_slow, cb_sem, Wg_sem, bg_sem)
    zq_syn = _sc_gather(cb_syn, idx_syn)
    zq_sem = _sc_gather(cb_sem, idx_sem)
    loss = (ls[0, 0] + lm[0, 0]) * ((1.0 + COMMITMENT_COST) / (N * D))
    zqc_syn = jax.lax.complex(zq_syn[:, :half], zq_syn[:, half:])
    zqc_sem = jax.lax.complex(zq_sem[:, :half], zq_sem[:, half:])
    return (zqc_syn, zqc_sem, loss, (idx_syn, idx_sem))


# speedup vs baseline: 1.2671x; 1.0023x over previous
import functools

import jax
import jax.numpy as jnp
from jax import lax
from jax.experimental import pallas as pl
from jax.experimental.pallas import tpu as pltpu
from jax.experimental.pallas import tpu_sc as plsc

GRAPH_BIAS_SCALE = 0.8
CONTEXT_GATE_STRENGTH = 2.0
COMMITMENT_COST = 0.25

N = 8192
D = 256
BN = 1024


def _body(z_ref, cb_ref, w_ref, b_ref, idx_ref, loss_ref, cb2_ref, cn_ref,
          iota4_ref):
    step = pl.program_id(0)

    @pl.when(step == 0)
    def _():
        loss_ref[0, 0] = 0.0
        cb = cb_ref[...]
        cb2_ref[...] = cb + cb
        cn_ref[...] = jnp.sum(cb * cb, axis=1)[None, :]
        iota4_ref[...] = (jax.lax.broadcasted_iota(
            jnp.int32, iota4_ref.shape, 1) * 4).astype(jnp.float32)

    z = z_ref[...]
    zn = jnp.sum(z * z, axis=1, keepdims=True)
    zc2 = jax.lax.dot_general(z, cb2_ref[...], (((1,), (1,)), ((), ())),
                              preferred_element_type=jnp.float32)
    logits = jnp.dot(z, w_ref[...], preferred_element_type=jnp.float32) + b_ref[...]
    e = jnp.exp(logits)
    g = (CONTEXT_GATE_STRENGTH / jnp.sum(e, axis=1, keepdims=True)) * e
    d = (zn + cn_ref[...] - zc2) - g
    k = d.shape[1]
    d_min = jnp.min(d, axis=1)
    eq = d == d_min[:, None]
    code = iota4_ref[...] - g
    v = jnp.min(jnp.where(eq, code, 4.0 * k), axis=1)
    idx_f = jnp.floor(v * 0.25 + 0.5)
    idx_ref[...] = idx_f.astype(jnp.int32)
    g_idx = idx_f * 4.0 - v
    loss_ref[0, 0] += jnp.sum(d_min + g_idx)


def _tc_branch(z, cb, w, b):
    k = cb.shape[0]
    grid = (N // BN,)
    full = lambda shape: pl.BlockSpec(shape, lambda i: (0,) * len(shape))

    idx, losssum = pl.pallas_call(
        _body,
        grid=grid,
        in_specs=[
            pl.BlockSpec((BN, D), lambda i: (i, 0)),
            full((k, D)),
            full((D, k)),
            full((1, k)),
        ],
        out_specs=(
            pl.BlockSpec((BN,), lambda i: (i,)),
            pl.BlockSpec((1, 1), lambda i: (0, 0), memory_space=pltpu.SMEM),
        ),
        out_shape=(
            jax.ShapeDtypeStruct((N,), jnp.int32),
            jax.ShapeDtypeStruct((1, 1), jnp.float32),
        ),
        scratch_shapes=[
            pltpu.VMEM((k, D), jnp.float32),
            pltpu.VMEM((1, k), jnp.float32),
            pltpu.VMEM((1, k), jnp.float32),
        ],
    )(z, cb, w, b.reshape(1, k))
    return idx, losssum


def _sc_gather(cb, idx):
    info = plsc.get_sparse_core_info()
    nw = info.num_cores * info.num_subcores
    b_per_w = N // nw
    mesh = plsc.VectorSubcoreMesh(core_axis_name="c", subcore_axis_name="s")

    @functools.partial(
        pl.kernel,
        mesh=mesh,
        out_type=jax.ShapeDtypeStruct((N, D), jnp.float32),
        scratch_types=[
            pltpu.VMEM((b_per_w,), jnp.int32),
            pltpu.VMEM((b_per_w, D), jnp.float32),
            pltpu.SemaphoreType.DMA,
        ],
    )
    def _k(cb_hbm, idx_hbm, out_hbm, idx_v, rows_v, sem):
        wid = lax.axis_index("s") * info.num_cores + lax.axis_index("c")
        base = wid * b_per_w
        pltpu.sync_copy(idx_hbm.at[pl.ds(base, b_per_w)], idx_v)
        pltpu.async_copy(cb_hbm.at[idx_v], rows_v, sem).wait()
        pltpu.sync_copy(rows_v, out_hbm.at[pl.ds(base, b_per_w)])

    return _k(cb, idx)


def kernel(z_fast, z_slow, cb_syn, cb_sem, Wg_syn, bg_syn, Wg_sem, bg_sem):
    half = D // 2
    idx_syn, ls = _tc_branch(z_fast, cb_syn, Wg_syn, bg_syn)
    idx_sem, lm = _tc_branch(z_slow, cb_sem, Wg_sem, bg_sem)
    zq_syn = _sc_gather(cb_syn, idx_syn)
    zq_sem = _sc_gather(cb_sem, idx_sem)
    loss = (ls[0, 0] + lm[0, 0]) * ((1.0 + COMMITMENT_COST) / (N * D))
    zqc_syn = jax.lax.complex(zq_syn[:, :half], zq_syn[:, half:])
    zqc_sem = jax.lax.complex(zq_sem[:, :half], zq_sem[:, half:])
    return (zqc_syn, zqc_sem, loss, (idx_syn, idx_sem))
